# baseline (speedup 1.0000x reference)
import functools

import jax
import jax.numpy as jnp
from jax.experimental import pallas as pl
from jax.experimental.pallas import tpu as pltpu

E = 16
D_STATE = 768
D_ACTION = 64
HIDDEN = 256
N_TOKENS = 16384
BLK = 256
NB = N_TOKENS // BLK + E
P = NB * BLK


def _routing_metadata(policy_indices):
    pol = policy_indices.astype(jnp.int32)
    order = jnp.argsort(pol, stable=True).astype(jnp.int32)
    counts = jnp.bincount(pol, length=E)
    off = jnp.cumsum(counts) - counts
    padded = ((counts + BLK - 1) // BLK) * BLK
    pad_off = (jnp.cumsum(padded) - padded).astype(jnp.int32)
    e_r = jnp.sort(pol)
    ranks = jnp.arange(N_TOKENS, dtype=jnp.int32)
    ppos = (pad_off[e_r] + (ranks - off[e_r])).astype(jnp.int32)
    src = jnp.zeros((P,), jnp.int32).at[ppos].set(order)
    inv = jnp.zeros((N_TOKENS,), jnp.int32).at[order].set(ppos)
    block_expert = jnp.clip(
        jnp.searchsorted(pad_off, jnp.arange(NB, dtype=jnp.int32) * BLK, side="right") - 1,
        0, E - 1).astype(jnp.int32)
    return src, inv, block_expert


def _mlp_body(be_ref, lat_ref, act_ref, w1s_ref, w1a_ref, b1_ref, w2_ref, b2_ref, out_ref):
    h = jnp.dot(lat_ref[...], w1s_ref[0], preferred_element_type=jnp.float32)
    h = h + jnp.dot(act_ref[...], w1a_ref[0], preferred_element_type=jnp.float32)
    h = jnp.maximum(h + b1_ref[0, 0], 0.0)
    out_ref[...] = jnp.dot(h, w2_ref[0], preferred_element_type=jnp.float32) + b2_ref[0, 0]


def _grouped_mlp(block_expert, lat_s, act_s, W1s, W1a, b1, W2, b2, interpret=False):
    grid_spec = pltpu.PrefetchScalarGridSpec(
        num_scalar_prefetch=1,
        grid=(NB,),
        in_specs=[
            pl.BlockSpec((BLK, D_STATE), lambda i, be: (i, 0)),
            pl.BlockSpec((BLK, D_ACTION), lambda i, be: (i, 0)),
            pl.BlockSpec((1, D_STATE, HIDDEN), lambda i, be: (be[i], 0, 0)),
            pl.BlockSpec((1, D_ACTION, HIDDEN), lambda i, be: (be[i], 0, 0)),
            pl.BlockSpec((1, 1, HIDDEN), lambda i, be: (be[i], 0, 0)),
            pl.BlockSpec((1, HIDDEN, D_STATE), lambda i, be: (be[i], 0, 0)),
            pl.BlockSpec((1, 1, D_STATE), lambda i, be: (be[i], 0, 0)),
        ],
        out_specs=pl.BlockSpec((BLK, D_STATE), lambda i, be: (i, 0)),
    )
    return pl.pallas_call(
        _mlp_body,
        grid_spec=grid_spec,
        out_shape=jax.ShapeDtypeStruct((P, D_STATE), jnp.float32),
        compiler_params=pltpu.CompilerParams(
            dimension_semantics=("arbitrary",),
        ),
        interpret=interpret,
    )(block_expert, lat_s, act_s, W1s, W1a, b1, W2, b2)


def kernel(latents, policy_indices, actions, W1, b1, W2, b2):
    src, inv, block_expert = _routing_metadata(policy_indices)
    lat_s = jnp.take(latents, src, axis=0)
    act_s = jnp.take(actions, src, axis=0)
    W1s = W1[:, :D_STATE, :]
    W1a = W1[:, D_STATE:, :]
    out_s = _grouped_mlp(block_expert, lat_s, act_s, W1s, W1a,
                         b1.reshape(E, 1, HIDDEN), W2, b2.reshape(E, 1, D_STATE))
    return jnp.take(out_s, inv, axis=0)


# speedup vs baseline: 1.3901x; 1.3901x over previous
import functools

import jax
import jax.numpy as jnp
from jax import lax
from jax.experimental import pallas as pl
from jax.experimental.pallas import tpu as pltpu
from jax.experimental.pallas import tpu_sc as plsc

E = 16
D_STATE = 768
D_ACTION = 64
HIDDEN = 256
N_TOKENS = 16384
BLK = 256
NB = N_TOKENS // BLK + E
P = NB * BLK
D_ACT_PAD = 128


def _routing_metadata(policy_indices):
    pol = policy_indices.astype(jnp.int32)
    order = jnp.argsort(pol, stable=True).astype(jnp.int32)
    counts = jnp.bincount(pol, length=E)
    off = jnp.cumsum(counts) - counts
    padded = ((counts + BLK - 1) // BLK) * BLK
    pad_off = (jnp.cumsum(padded) - padded).astype(jnp.int32)
    e_r = jnp.sort(pol)
    ranks = jnp.arange(N_TOKENS, dtype=jnp.int32)
    ppos = (pad_off[e_r] + (ranks - off[e_r])).astype(jnp.int32)
    src = jnp.zeros((P,), jnp.int32).at[ppos].set(order)
    inv = jnp.zeros((N_TOKENS,), jnp.int32).at[order].set(ppos)
    block_expert = jnp.clip(
        jnp.searchsorted(pad_off, jnp.arange(NB, dtype=jnp.int32) * BLK, side="right") - 1,
        0, E - 1).astype(jnp.int32)
    return src, inv, block_expert


NC = 2
NS = 16
NW = NC * NS


def _gather_in_body(src_hbm, lat_hbm, act_hbm, lat_out, act_out,
                    idx_v, lat_v, act_v, sem1, sem2):
    wid = lax.axis_index("s") * NC + lax.axis_index("c")
    rows = P // NW
    ch = 128
    base = wid * rows
    for c in range(rows // ch):
        b = base + c * ch
        pltpu.sync_copy(src_hbm.at[pl.ds(b, ch)], idx_v)
        d1 = pltpu.async_copy(lat_hbm.at[idx_v], lat_v, sem1)
        d2 = pltpu.async_copy(act_hbm.at[idx_v], act_v, sem2)
        d1.wait()
        d2.wait()
        pltpu.sync_copy(lat_v, lat_out.at[pl.ds(b, ch)])
        pltpu.sync_copy(act_v, act_out.at[pl.ds(b, ch)])


def _gather_inputs(src, latents, actions):
    ch = 128
    fn = pl.kernel(
        _gather_in_body,
        out_type=(jax.ShapeDtypeStruct((P, D_STATE), jnp.float32),
                  jax.ShapeDtypeStruct((P, D_ACT_PAD), jnp.float32)),
        mesh=plsc.VectorSubcoreMesh(core_axis_name="c", subcore_axis_name="s"),
        scratch_types=[
            pltpu.VMEM((ch,), jnp.int32),
            pltpu.VMEM((ch, D_STATE), jnp.float32),
            pltpu.VMEM((ch, D_ACT_PAD), jnp.float32),
            pltpu.SemaphoreType.DMA,
            pltpu.SemaphoreType.DMA,
        ],
    )
    return fn(src, latents, actions)


def _gather_out_body(inv_hbm, outs_hbm, out_hbm, idx_v, rows_v, sem):
    wid = lax.axis_index("s") * NC + lax.axis_index("c")
    rows = N_TOKENS // NW
    ch = 128
    base = wid * rows
    for c in range(rows // ch):
        b = base + c * ch
        pltpu.sync_copy(inv_hbm.at[pl.ds(b, ch)], idx_v)
        pltpu.async_copy(outs_hbm.at[idx_v], rows_v, sem).wait()
        pltpu.sync_copy(rows_v, out_hbm.at[pl.ds(b, ch)])


def _gather_output(inv, out_s):
    ch = 128
    fn = pl.kernel(
        _gather_out_body,
        out_type=jax.ShapeDtypeStruct((N_TOKENS, D_STATE), jnp.float32),
        mesh=plsc.VectorSubcoreMesh(core_axis_name="c", subcore_axis_name="s"),
        scratch_types=[
            pltpu.VMEM((ch,), jnp.int32),
            pltpu.VMEM((ch, D_STATE), jnp.float32),
            pltpu.SemaphoreType.DMA,
        ],
    )
    return fn(inv, out_s)


def _mlp_body(be_ref, lat_ref, act_ref, w1s_ref, w1a_ref, b1_ref, w2_ref, b2_ref, out_ref):
    h = jnp.dot(lat_ref[...], w1s_ref[0], preferred_element_type=jnp.float32)
    h = h + jnp.dot(act_ref[...], w1a_ref[0], preferred_element_type=jnp.float32)
    h = jnp.maximum(h + b1_ref[0, 0], 0.0)
    out_ref[...] = jnp.dot(h, w2_ref[0], preferred_element_type=jnp.float32) + b2_ref[0, 0]


def _grouped_mlp(block_expert, lat_s, act_s, W1s, W1a, b1, W2, b2, interpret=False):
    grid_spec = pltpu.PrefetchScalarGridSpec(
        num_scalar_prefetch=1,
        grid=(NB,),
        in_specs=[
            pl.BlockSpec((BLK, D_STATE), lambda i, be: (i, 0)),
            pl.BlockSpec((BLK, D_ACT_PAD), lambda i, be: (i, 0)),
            pl.BlockSpec((1, D_STATE, HIDDEN), lambda i, be: (be[i], 0, 0)),
            pl.BlockSpec((1, D_ACT_PAD, HIDDEN), lambda i, be: (be[i], 0, 0)),
            pl.BlockSpec((1, 1, HIDDEN), lambda i, be: (be[i], 0, 0)),
            pl.BlockSpec((1, HIDDEN, D_STATE), lambda i, be: (be[i], 0, 0)),
            pl.BlockSpec((1, 1, D_STATE), lambda i, be: (be[i], 0, 0)),
        ],
        out_specs=pl.BlockSpec((BLK, D_STATE), lambda i, be: (i, 0)),
    )
    return pl.pallas_call(
        _mlp_body,
        grid_spec=grid_spec,
        out_shape=jax.ShapeDtypeStruct((P, D_STATE), jnp.float32),
        compiler_params=pltpu.CompilerParams(
            dimension_semantics=("arbitrary",),
        ),
        interpret=interpret,
    )(block_expert, lat_s, act_s, W1s, W1a, b1, W2, b2)


def kernel(latents, policy_indices, actions, W1, b1, W2, b2):
    src, inv, block_expert = _routing_metadata(policy_indices)
    actions_pad = jnp.pad(actions, ((0, 0), (0, D_ACT_PAD - D_ACTION)))
    lat_s, act_s = _gather_inputs(src, latents, actions_pad)
    W1s = W1[:, :D_STATE, :]
    W1a = jnp.pad(W1[:, D_STATE:, :], ((0, 0), (0, D_ACT_PAD - D_ACTION), (0, 0)))
    out_s = _grouped_mlp(block_expert, lat_s, act_s, W1s, W1a,
                         b1.reshape(E, 1, HIDDEN), W2, b2.reshape(E, 1, D_STATE))
    return _gather_output(inv, out_s)
